# Initial kernel scaffold; baseline (speedup 1.0000x reference)
import jax
import jax.numpy as jnp
from jax.experimental import pallas as pl


def kernel(x, log_alpha, log_delta, log_r):
    raise NotImplementedError("write your pallas kernel here")


# speedup vs baseline: 37.9952x; 37.9952x over previous
import numpy as np
import jax
import jax.numpy as jnp
from jax.experimental import pallas as pl
from jax.experimental.pallas import tpu as pltpu

_T_VALUES = (2.0, 8.0, 32.0, 128.0)
_EPS = 1e-05
_W = 256


def _s_vals():
    t = np.asarray(_T_VALUES, dtype=np.float64)
    return (np.sqrt(1.0 + 4.0 * t * t) - 1.0) / (2.0 * t * t)


_S = _s_vals()


def _tri_matrices():
    i = np.arange(_W)[:, None]
    j = np.arange(_W)[None, :]
    d = np.maximum(j - i, 0)
    mats = []
    for s in _S:
        u = np.where(j >= i, s * np.exp(np.log1p(-s) * d), 0.0)
        mats.append(u)
    return np.stack(mats).astype(np.float32)


_U_CONST = _tri_matrices()


def _mrpcen_body(x_ref, u_ref, p_ref, o_ref, carry_ref):
    k = pl.program_id(1)
    xb = x_ref[0]
    F = xb.shape[0]

    @pl.when(k == 0)
    def _():
        carry_ref[...] = jnp.broadcast_to(xb[:, 0:1], carry_ref.shape)

    col0 = jax.lax.broadcasted_iota(jnp.int32, (1, _W), 1) == 0
    alpha = p_ref[:, 0:1]
    delta = p_ref[:, 1:2]
    r = p_ref[:, 2:3]
    dr = p_ref[:, 3:4]

    for t in range(4):
        s = float(_S[t])
        g = (1.0 - s) / s
        c = carry_ref[:, t : t + 1]
        xq = xb + jnp.where(col0, g * c, 0.0)
        m = jax.lax.dot_general(
            xq,
            u_ref[t],
            (((1,), (0,)), ((), ())),
            preferred_element_type=jnp.float32,
        )
        carry_ref[:, t : t + 1] = m[:, _W - 1 : _W]
        smooth = jnp.exp(-alpha * jnp.log(_EPS + m))
        o_ref[0, t] = jnp.exp(r * jnp.log(xb * smooth + delta)) - dr


def kernel(x, log_alpha, log_delta, log_r):
    B, F, N = x.shape
    alpha = jnp.exp(log_alpha)
    delta = jnp.exp(log_delta)
    r = jnp.exp(log_r)
    dr = delta**r
    zeros = jnp.zeros_like(alpha)
    params = jnp.stack([alpha, delta, r, dr, zeros, zeros, zeros, zeros], axis=1)
    u = jnp.asarray(_U_CONST)

    return pl.pallas_call(
        _mrpcen_body,
        grid=(B, N // _W),
        in_specs=[
            pl.BlockSpec((1, F, _W), lambda b, k: (b, 0, k)),
            pl.BlockSpec((4, _W, _W), lambda b, k: (0, 0, 0)),
            pl.BlockSpec((F, 8), lambda b, k: (0, 0)),
        ],
        out_specs=pl.BlockSpec((1, 4, F, _W), lambda b, k: (b, 0, 0, k)),
        out_shape=jax.ShapeDtypeStruct((B, 4, F, N), x.dtype),
        scratch_shapes=[pltpu.VMEM((F, 8), jnp.float32)],
        compiler_params=pltpu.CompilerParams(
            dimension_semantics=("parallel", "arbitrary"),
        ),
    )(x, u, params)


# speedup vs baseline: 55.7973x; 1.4685x over previous
import numpy as np
import jax
import jax.numpy as jnp
from jax.experimental import pallas as pl
from jax.experimental.pallas import tpu as pltpu

_T_VALUES = (2.0, 8.0, 32.0, 128.0)
_EPS = 1e-05
_W = 256


def _s_vals():
    t = np.asarray(_T_VALUES, dtype=np.float64)
    return (np.sqrt(1.0 + 4.0 * t * t) - 1.0) / (2.0 * t * t)


_S = _s_vals()


def _tri_matrix_cat():
    i = np.arange(_W)[:, None]
    j = np.arange(_W)[None, :]
    d = np.maximum(j - i, 0)
    mats = []
    for s in _S:
        u = np.where(j >= i, s * np.exp(np.log1p(-s) * d), 0.0)
        mats.append(u)
    return np.concatenate(mats, axis=1).astype(np.float32)


_U_CAT = _tri_matrix_cat()


def _mrpcen_body(x_ref, u_ref, p_ref, o_ref, carry_ref):
    k = pl.program_id(1)
    xb = x_ref[0]

    @pl.when(k == 0)
    def _():
        carry_ref[...] = jnp.broadcast_to(xb[:, 0:1], carry_ref.shape)

    c_all = carry_ref[...]
    alpha = p_ref[:, 0:1]
    delta = p_ref[:, 1:2]
    r = p_ref[:, 2:3]
    dr = p_ref[:, 3:4]

    m_all = jax.lax.dot_general(
        xb,
        u_ref[...],
        (((1,), (0,)), ((), ())),
        preferred_element_type=jnp.float32,
    )

    new_c = []
    for t in range(4):
        s = float(_S[t])
        g = (1.0 - s) / s
        d = u_ref[0:1, t * _W : (t + 1) * _W] * g
        c = c_all[:, t : t + 1]
        m = m_all[:, t * _W : (t + 1) * _W] + c * d
        new_c.append(m[:, _W - 1 : _W])
        smooth = jnp.exp2(-alpha * jnp.log2(_EPS + m))
        o_ref[0, t] = jnp.exp2(r * jnp.log2(xb * smooth + delta)) - dr

    carry_ref[:, 0:4] = jnp.concatenate(new_c, axis=1)


def kernel(x, log_alpha, log_delta, log_r):
    B, F, N = x.shape
    alpha = jnp.exp(log_alpha)
    delta = jnp.exp(log_delta)
    r = jnp.exp(log_r)
    dr = delta**r
    zeros = jnp.zeros_like(alpha)
    params = jnp.stack([alpha, delta, r, dr, zeros, zeros, zeros, zeros], axis=1)
    u = jnp.asarray(_U_CAT)

    return pl.pallas_call(
        _mrpcen_body,
        grid=(B, N // _W),
        in_specs=[
            pl.BlockSpec((1, F, _W), lambda b, k: (b, 0, k)),
            pl.BlockSpec((_W, 4 * _W), lambda b, k: (0, 0)),
            pl.BlockSpec((F, 8), lambda b, k: (0, 0)),
        ],
        out_specs=pl.BlockSpec((1, 4, F, _W), lambda b, k: (b, 0, 0, k)),
        out_shape=jax.ShapeDtypeStruct((B, 4, F, N), x.dtype),
        scratch_shapes=[pltpu.VMEM((F, 8), jnp.float32)],
        compiler_params=pltpu.CompilerParams(
            dimension_semantics=("parallel", "arbitrary"),
        ),
    )(x, u, params)


# speedup vs baseline: 100.0256x; 1.7927x over previous
import numpy as np
import jax
import jax.numpy as jnp
from jax.experimental import pallas as pl
from jax.experimental.pallas import tpu as pltpu

_T_VALUES = (2.0, 8.0, 32.0, 128.0)
_EPS = 1e-05
_W = 256


def _s_vals():
    t = np.asarray(_T_VALUES, dtype=np.float64)
    return (np.sqrt(1.0 + 4.0 * t * t) - 1.0) / (2.0 * t * t)


_S = _s_vals()


def _tri_matrix_cat():
    i = np.arange(_W)[:, None]
    j = np.arange(_W)[None, :]
    d = np.maximum(j - i, 0)
    mats = []
    for s in _S:
        u = np.where(j >= i, s * np.exp(np.log1p(-s) * d), 0.0)
        mats.append(u)
    return np.concatenate(mats, axis=1).astype(np.float32)


_U_CAT = _tri_matrix_cat()


_G = 4


def _mrpcen_body(x_ref, u_ref, p_ref, o_ref, carry_ref):
    k = pl.program_id(1)
    gf = x_ref.shape[0] * x_ref.shape[1]
    w = x_ref.shape[2]
    xb = x_ref[...].reshape(gf, w)

    @pl.when(k == 0)
    def _():
        carry_ref[...] = jnp.broadcast_to(xb[:, 0:1], carry_ref.shape)

    c_all = carry_ref[...]
    alpha = p_ref[:, 0:1]
    delta = p_ref[:, 1:2]
    r = p_ref[:, 2:3]
    dr = p_ref[:, 3:4]

    m_all = jax.lax.dot_general(
        xb,
        u_ref[...],
        (((1,), (0,)), ((), ())),
        preferred_element_type=jnp.float32,
    )

    new_c = []
    for t in range(4):
        s = float(_S[t])
        g = (1.0 - s) / s
        d = u_ref[0:1, t * _W : (t + 1) * _W] * g
        c = c_all[:, t : t + 1]
        m = m_all[:, t * _W : (t + 1) * _W] + c * d
        new_c.append(m[:, _W - 1 : _W])
        smooth = jnp.exp2(-alpha * jnp.log2(_EPS + m))
        pcen = jnp.exp2(r * jnp.log2(xb * smooth + delta)) - dr
        o_ref[:, t] = pcen.reshape(x_ref.shape[0], x_ref.shape[1], w)

    carry_ref[:, 0:4] = jnp.concatenate(new_c, axis=1)


def kernel(x, log_alpha, log_delta, log_r):
    B, F, N = x.shape
    alpha = jnp.exp(log_alpha)
    delta = jnp.exp(log_delta)
    r = jnp.exp(log_r)
    dr = delta**r
    zeros = jnp.zeros_like(alpha)
    params = jnp.stack([alpha, delta, r, dr, zeros, zeros, zeros, zeros], axis=1)
    params = jnp.tile(params, (_G, 1))
    u = jnp.asarray(_U_CAT)

    return pl.pallas_call(
        _mrpcen_body,
        grid=(B // _G, N // _W),
        in_specs=[
            pl.BlockSpec((_G, F, _W), lambda b, k: (b, 0, k)),
            pl.BlockSpec((_W, 4 * _W), lambda b, k: (0, 0)),
            pl.BlockSpec((_G * F, 8), lambda b, k: (0, 0)),
        ],
        out_specs=pl.BlockSpec((_G, 4, F, _W), lambda b, k: (b, 0, 0, k)),
        out_shape=jax.ShapeDtypeStruct((B, 4, F, N), x.dtype),
        scratch_shapes=[pltpu.VMEM((_G * F, 8), jnp.float32)],
        compiler_params=pltpu.CompilerParams(
            dimension_semantics=("parallel", "arbitrary"),
        ),
    )(x, u, params)


# speedup vs baseline: 111.5435x; 1.1151x over previous
import numpy as np
import jax
import jax.numpy as jnp
from jax.experimental import pallas as pl
from jax.experimental.pallas import tpu as pltpu

_T_VALUES = (2.0, 8.0, 32.0, 128.0)
_EPS = 1e-05
_W = 256


def _s_vals():
    t = np.asarray(_T_VALUES, dtype=np.float64)
    return (np.sqrt(1.0 + 4.0 * t * t) - 1.0) / (2.0 * t * t)


_S = _s_vals()


def _tri_matrix_cat():
    i = np.arange(_W)[:, None]
    j = np.arange(_W)[None, :]
    d = np.maximum(j - i, 0)
    mats = []
    for s in _S:
        u = np.where(j >= i, s * np.exp(np.log1p(-s) * d), 0.0)
        mats.append(u)
    return np.concatenate(mats, axis=1).astype(np.float32)


_U_CAT = _tri_matrix_cat()


def _mrpcen_body(x_ref, u_ref, p_ref, o_ref, carry_ref):
    k = pl.program_id(0)
    gdim, fdim, w = x_ref.shape
    rows = gdim * fdim
    xb = x_ref[...].reshape(rows, w)

    @pl.when(k == 0)
    def _():
        carry_ref[...] = jnp.broadcast_to(xb[:, 0:1], carry_ref.shape)

    c_all = carry_ref[...]
    alpha = p_ref[0]
    delta = p_ref[1]
    r = p_ref[2]
    dr = p_ref[3]

    m_all = jax.lax.dot_general(
        xb,
        u_ref[...],
        (((1,), (0,)), ((), ())),
        preferred_element_type=jnp.float32,
    )

    new_c = []
    for t in range(4):
        s = float(_S[t])
        g = (1.0 - s) / s
        d = u_ref[0:1, t * _W : (t + 1) * _W] * g
        c = c_all[:, t : t + 1]
        m = m_all[:, t * _W : (t + 1) * _W] + c * d
        new_c.append(m[:, _W - 1 : _W])
        smooth = jnp.exp2(-alpha * jnp.log2(_EPS + m))
        pcen = jnp.exp2(r * jnp.log2(xb * smooth + delta)) - dr
        o_ref[:, t] = pcen.reshape(gdim, fdim, w)

    carry_ref[:, 0:4] = jnp.concatenate(new_c, axis=1)


def kernel(x, log_alpha, log_delta, log_r):
    B, F, N = x.shape
    rows = B * F
    alpha = jnp.exp(log_alpha)
    delta = jnp.exp(log_delta)
    r = jnp.exp(log_r)
    dr = delta**r
    params = jnp.stack([alpha, delta, r, dr])
    params = jnp.broadcast_to(params[:, None, :, None], (4, B, F, _W))
    params = params.reshape(4, rows, _W)
    u = jnp.asarray(_U_CAT)

    return pl.pallas_call(
        _mrpcen_body,
        grid=(N // _W,),
        in_specs=[
            pl.BlockSpec((B, F, _W), lambda k: (0, 0, k)),
            pl.BlockSpec((_W, 4 * _W), lambda k: (0, 0)),
            pl.BlockSpec((4, rows, _W), lambda k: (0, 0, 0)),
        ],
        out_specs=pl.BlockSpec((B, 4, F, _W), lambda k: (0, 0, 0, k)),
        out_shape=jax.ShapeDtypeStruct((B, 4, F, N), x.dtype),
        scratch_shapes=[pltpu.VMEM((rows, 8), jnp.float32)],
        compiler_params=pltpu.CompilerParams(
            dimension_semantics=("arbitrary",),
        ),
    )(x, u, params)


# speedup vs baseline: 117.3740x; 1.0523x over previous
import numpy as np
import jax
import jax.numpy as jnp
from jax.experimental import pallas as pl
from jax.experimental.pallas import tpu as pltpu

_T_VALUES = (2.0, 8.0, 32.0, 128.0)
_EPS = 1e-05
_W = 256


def _s_vals():
    t = np.asarray(_T_VALUES, dtype=np.float64)
    return (np.sqrt(1.0 + 4.0 * t * t) - 1.0) / (2.0 * t * t)


_S = _s_vals()


def _aug_matrix():
    i = np.arange(_W)[:, None]
    j = np.arange(_W)[None, :]
    d = np.maximum(j - i, 0)
    u = np.zeros((2 * _W, 4 * _W), dtype=np.float64)
    for t, s in enumerate(_S):
        sl = slice(t * _W, (t + 1) * _W)
        u[:_W, sl] = np.where(j >= i, s * np.exp(np.log1p(-s) * d), 0.0)
        dvec = np.exp(np.log1p(-s) * (np.arange(_W) + 1.0))
        u[_W + t, sl] = dvec
        u[_W + 4, sl] = _EPS * (1.0 - dvec)
    return u.astype(np.float32)


_U_AUG = _aug_matrix()


def _mrpcen_body(x_ref, u_ref, p_ref, o_ref, carry_ref):
    k = pl.program_id(0)
    gdim, fdim, w = x_ref.shape
    rows = gdim * fdim
    xb = x_ref[...].reshape(rows, w)

    @pl.when(k == 0)
    def _():
        lane = jax.lax.broadcasted_iota(jnp.int32, (rows, _W), 1)
        carry_ref[...] = jnp.where(
            lane < 4,
            xb[:, 0:1] + _EPS,
            jnp.where(lane == 4, 1.0, 0.0),
        )

    lhs = jnp.concatenate([xb, carry_ref[...]], axis=1)
    nalpha = p_ref[0]
    delta = p_ref[1]
    r = p_ref[2]
    dr = p_ref[3]

    me_all = jax.lax.dot_general(
        lhs,
        u_ref[...],
        (((1,), (0,)), ((), ())),
        preferred_element_type=jnp.float32,
    )

    new_c = []
    for t in range(4):
        me = me_all[:, t * _W : (t + 1) * _W]
        new_c.append(me[:, _W - 1 : _W])
        smooth = jnp.exp2(nalpha * jnp.log2(me))
        pcen = jnp.exp2(r * jnp.log2(xb * smooth + delta)) - dr
        o_ref[:, t] = pcen.reshape(gdim, fdim, w)

    carry_ref[:, 0:4] = jnp.concatenate(new_c, axis=1)


def kernel(x, log_alpha, log_delta, log_r):
    B, F, N = x.shape
    rows = B * F
    alpha = jnp.exp(log_alpha)
    delta = jnp.exp(log_delta)
    r = jnp.exp(log_r)
    dr = delta**r
    params = jnp.stack([-alpha, delta, r, dr])
    params = jnp.broadcast_to(params[:, None, :, None], (4, B, F, _W))
    params = params.reshape(4, rows, _W)
    u = jnp.asarray(_U_AUG)

    return pl.pallas_call(
        _mrpcen_body,
        grid=(N // _W,),
        in_specs=[
            pl.BlockSpec((B, F, _W), lambda k: (0, 0, k)),
            pl.BlockSpec((2 * _W, 4 * _W), lambda k: (0, 0)),
            pl.BlockSpec((4, rows, _W), lambda k: (0, 0, 0)),
        ],
        out_specs=pl.BlockSpec((B, 4, F, _W), lambda k: (0, 0, 0, k)),
        out_shape=jax.ShapeDtypeStruct((B, 4, F, N), x.dtype),
        scratch_shapes=[pltpu.VMEM((rows, _W), jnp.float32)],
        compiler_params=pltpu.CompilerParams(
            dimension_semantics=("arbitrary",),
        ),
    )(x, u, params)


# speedup vs baseline: 118.1495x; 1.0066x over previous
import numpy as np
import jax
import jax.numpy as jnp
from jax.experimental import pallas as pl
from jax.experimental.pallas import tpu as pltpu

_T_VALUES = (2.0, 8.0, 32.0, 128.0)
_EPS = 1e-05
_W = 256


def _s_vals():
    t = np.asarray(_T_VALUES, dtype=np.float64)
    return (np.sqrt(1.0 + 4.0 * t * t) - 1.0) / (2.0 * t * t)


_S = _s_vals()


def _aug_matrix():
    i = np.arange(_W)[:, None]
    j = np.arange(_W)[None, :]
    d = np.maximum(j - i, 0)
    u = np.zeros((2 * _W, 4 * _W), dtype=np.float64)
    for t, s in enumerate(_S):
        sl = slice(t * _W, (t + 1) * _W)
        u[:_W, sl] = np.where(j >= i, s * np.exp(np.log1p(-s) * d), 0.0)
        dvec = np.exp(np.log1p(-s) * (np.arange(_W) + 1.0))
        u[_W + t, sl] = dvec
        u[_W + 4, sl] = _EPS * (1.0 - dvec)
    return u.astype(np.float32)


_U_AUG = _aug_matrix()


def _mrpcen_body(x_ref, u_ref, p_ref, o_ref, carry_ref):
    k = pl.program_id(0)
    gdim, fdim, w = x_ref.shape
    rows = gdim * fdim
    xb = x_ref[...].reshape(rows, w)

    @pl.when(k == 0)
    def _():
        lane = jax.lax.broadcasted_iota(jnp.int32, (rows, _W), 1)
        carry_ref[...] = jnp.where(
            lane < 4,
            xb[:, 0:1] + _EPS,
            jnp.where(lane == 4, 1.0, 0.0),
        )

    lhs = jnp.concatenate([xb, carry_ref[...]], axis=1)
    nalpha2 = p_ref[0]
    delta = p_ref[1]
    r2 = p_ref[2]
    dr = p_ref[3]

    me_all = jax.lax.dot_general(
        lhs,
        u_ref[...],
        (((1,), (0,)), ((), ())),
        preferred_element_type=jnp.float32,
    )

    new_c = []
    for t in range(4):
        me = me_all[:, t * _W : (t + 1) * _W]
        new_c.append(me[:, _W - 1 : _W])
        smooth = jnp.exp2(nalpha2 * jnp.log(me))
        pcen = jnp.exp2(r2 * jnp.log(xb * smooth + delta)) - dr
        o_ref[:, t] = pcen.reshape(gdim, fdim, w)

    carry_ref[:, 0:4] = jnp.concatenate(new_c, axis=1)


def kernel(x, log_alpha, log_delta, log_r):
    B, F, N = x.shape
    rows = B * F
    alpha = jnp.exp(log_alpha)
    delta = jnp.exp(log_delta)
    r = jnp.exp(log_r)
    dr = delta**r
    inv_ln2 = float(1.0 / np.log(2.0))
    params = jnp.stack([-alpha * inv_ln2, delta, r * inv_ln2, dr])
    params = jnp.broadcast_to(params[:, None, :, None], (4, B, F, _W))
    params = params.reshape(4, rows, _W)
    u = jnp.asarray(_U_AUG)

    return pl.pallas_call(
        _mrpcen_body,
        grid=(N // _W,),
        in_specs=[
            pl.BlockSpec((B, F, _W), lambda k: (0, 0, k)),
            pl.BlockSpec((2 * _W, 4 * _W), lambda k: (0, 0)),
            pl.BlockSpec((4, rows, _W), lambda k: (0, 0, 0)),
        ],
        out_specs=pl.BlockSpec((B, 4, F, _W), lambda k: (0, 0, 0, k)),
        out_shape=jax.ShapeDtypeStruct((B, 4, F, N), x.dtype),
        scratch_shapes=[pltpu.VMEM((rows, _W), jnp.float32)],
        compiler_params=pltpu.CompilerParams(
            dimension_semantics=("arbitrary",),
        ),
    )(x, u, params)
